# baseline (speedup 1.0000x reference)
import jax
import jax.numpy as jnp
from jax import lax
from jax.experimental import pallas as pl
from jax.experimental.pallas import tpu as pltpu
from jax.experimental.pallas import tpu_sc as plsc

B = 4
N = 8192
H = 768
C = 256
K = 32

TN = 2048
NT = N // TN

L = 16
NWORK = 32
CHUNKS_PER_W = (B * C // L) // NWORK


def _enc_body(x_ref, w_ref, b_ref, wa_ref, ba_ref, u_ref, z_ref, psum):
    bi = pl.program_id(0)
    n = pl.program_id(1)
    xt = x_ref[0]
    u = jnp.maximum(
        jnp.dot(xt, w_ref[...], preferred_element_type=jnp.float32)
        + b_ref[...], 0.0)
    u_ref[0] = u
    part = jnp.sum(u, axis=0, keepdims=True)

    @pl.when(n == 0)
    def _():
        psum[pl.ds(bi, 1), :] = part

    @pl.when(n != 0)
    def _():
        psum[pl.ds(bi, 1), :] += part

    @pl.when((bi == B - 1) & (n == NT - 1))
    def _agg():
        u_bar = psum[...] * (1.0 / N)
        logits = jnp.dot(u_bar, wa_ref[...],
                         preferred_element_type=jnp.float32) + ba_ref[...]
        z_ref[...] = jax.nn.sigmoid(logits)


def _sc_topk_body(z_hbm, o_hbm, zv, ov, sem):
    wid = lax.axis_index("s") * 2 + lax.axis_index("c")
    b = wid // 8
    base = (wid % 8) * (CHUNKS_PER_W * L)
    pltpu.async_copy(z_hbm.at[b], zv, sem).wait()

    for ci in range(CHUNKS_PER_W):
        off = base + ci * L
        v = zv[pl.ds(off, L)]
        gidx = jax.lax.broadcasted_iota(jnp.int32, (L,), 0) + off
        zero = jnp.zeros((L,), jnp.int32)
        one = jnp.ones((L,), jnp.int32)

        def body(jc, cnt):
            w = zv[pl.ds(jc * L, L)]
            for k in range(L):
                s = w[k]
                c = jc * L + k
                gt = jnp.where(s > v, one, zero)
                tie = jnp.where((s == v) & (c < gidx), one, zero)
                cnt = cnt + gt + tie
            return cnt

        cnt = lax.fori_loop(0, C // L, body, zero)
        ov[pl.ds(ci * L, L)] = jnp.where(cnt < K, v, jnp.zeros((L,), jnp.float32))

    pltpu.async_copy(ov, o_hbm.at[b, pl.ds(base, CHUNKS_PER_W * L)], sem).wait()


def _dec_body(u_ref, z_ref, w_ref, b_ref, o_ref):
    bi = pl.program_id(0)
    u = u_ref[0]
    gated = u * z_ref[pl.ds(bi, 1), :]
    o = jnp.dot(gated, w_ref[...], preferred_element_type=jnp.float32)
    o_ref[0] = o + b_ref[...]


def _sc_topk(z):
    kern = pl.kernel(
        _sc_topk_body,
        out_type=jax.ShapeDtypeStruct((B, C), jnp.float32),
        mesh=plsc.VectorSubcoreMesh(core_axis_name="c", subcore_axis_name="s"),
        scratch_types=[
            pltpu.VMEM((C,), jnp.float32),
            pltpu.VMEM((CHUNKS_PER_W * L,), jnp.float32),
            pltpu.SemaphoreType.DMA,
        ],
    )
    return kern(z)


@jax.jit
def kernel(x, W_enc, b_enc, W_agg, b_agg, W_dec, b_dec):
    b_enc2 = b_enc.reshape(1, C)
    b_agg2 = b_agg.reshape(1, C)
    b_dec2 = b_dec.reshape(1, H)

    u, z = pl.pallas_call(
        _enc_body,
        grid=(B, NT),
        in_specs=[
            pl.BlockSpec((1, TN, H), lambda b, n: (b, n, 0)),
            pl.BlockSpec((H, C), lambda b, n: (0, 0)),
            pl.BlockSpec((1, C), lambda b, n: (0, 0)),
            pl.BlockSpec((C, C), lambda b, n: (0, 0)),
            pl.BlockSpec((1, C), lambda b, n: (0, 0)),
        ],
        out_specs=[
            pl.BlockSpec((1, TN, C), lambda b, n: (b, n, 0)),
            pl.BlockSpec((B, C), lambda b, n: (0, 0)),
        ],
        out_shape=[
            jax.ShapeDtypeStruct((B, N, C), jnp.float32),
            jax.ShapeDtypeStruct((B, C), jnp.float32),
        ],
        scratch_shapes=[pltpu.VMEM((B, C), jnp.float32)],
        compiler_params=pltpu.CompilerParams(
            dimension_semantics=("arbitrary", "arbitrary"),
        ),
    )(x, W_enc, b_enc2, W_agg, b_agg2)

    z_hat = _sc_topk(z)

    f_hat = pl.pallas_call(
        _dec_body,
        grid=(B, NT),
        in_specs=[
            pl.BlockSpec((1, TN, C), lambda b, n: (b, n, 0)),
            pl.BlockSpec((B, C), lambda b, n: (0, 0)),
            pl.BlockSpec((C, H), lambda b, n: (0, 0)),
            pl.BlockSpec((1, H), lambda b, n: (0, 0)),
        ],
        out_specs=pl.BlockSpec((1, TN, H), lambda b, n: (b, n, 0)),
        out_shape=jax.ShapeDtypeStruct((B, N, H), jnp.float32),
    )(u, z_hat, W_dec, b_dec2)

    return (f_hat, z_hat, u)
